# baseline (speedup 1.0000x reference)
import math

import jax
import jax.numpy as jnp
from jax import lax
from jax.experimental import pallas as pl
from jax.experimental.pallas import tpu as pltpu

N = 2048
BI = 32
BJ = 128
NBLK = N // BI
SCALE = 1.0 / math.sqrt(2.0)

_CT1 = (((1,), (1,)), ((), ()))


def _lrelu(x):
    return jnp.where(x >= 0, x, 0.2 * x)


def _group_norm_rows(x, gamma, beta, groups, eps=1e-5):
    c = x.shape[0]
    per = c // groups
    outs = []
    for g in range(groups):
        sub = x[g * per:(g + 1) * per, :]
        m = jnp.mean(sub)
        v = jnp.mean((sub - m) ** 2)
        outs.append((sub - m) / jnp.sqrt(v + eps))
    y = jnp.concatenate(outs, axis=0)
    return y * gamma + beta


def _fused_body(feat_ref, pts_full_ref, nuv_full_ref, pts_blk_ref, nuv_blk_ref,
                selx_ref, seln_ref, eye8_ref, eye16_ref,
                w_in1_ref, b_in1_ref, w_in2_ref, b_in2_ref, g_in_ref, be_in_ref,
                w1k_ref, w2k_ref,
                w_out1_ref, b_out1_ref, w_out2_ref, b_out2_ref,
                g_out_ref, be_out_ref,
                out_ref, jd_ref, pair_ref):
    pid = pl.program_id(0)

    @pl.when(pid == 0)
    def _prologue():
        f = lax.dot_general(w_in1_ref[...], feat_ref[...], _CT1,
                            preferred_element_type=jnp.float32) + b_in1_ref[...]
        f = _lrelu(f)
        f = jnp.dot(w_in2_ref[...], f,
                    preferred_element_type=jnp.float32) + b_in2_ref[...]
        f = _lrelu(f)
        f = _group_norm_rows(f, g_in_ref[...], be_in_ref[...], groups=4)
        xt = lax.dot_general(selx_ref[...], pts_full_ref[...], _CT1,
                             preferred_element_type=jnp.float32)
        nt = lax.dot_general(seln_ref[...], nuv_full_ref[...], _CT1,
                             preferred_element_type=jnp.float32)
        rows = ([xt[c:c + 1, :] for c in range(3)]
                + [nt[c:c + 1, :] for c in range(3)]
                + [f[h:h + 1, :] for h in range(8)])
        jd_ref[...] = jnp.concatenate(
            [jnp.broadcast_to(r, (BI, N)) for r in rows], axis=0)

    @pl.when((pid >= 1) & (pid <= NBLK))
    def _pair():
        xi = [pts_blk_ref[:, c:c + 1] * SCALE for c in range(3)]
        frame = [[nuv_blk_ref[:, 3 * k + c:3 * k + c + 1] for c in range(3)]
                 for k in range(3)]
        ni = frame[0]
        w1k = w1k_ref[...]
        w2k = w2k_ref[...]
        ones_b = jnp.ones((BI, BJ), jnp.bfloat16)
        nchunk = N // BJ
        ws, xss = [], []
        for blk in range(nchunk):
            sl = slice(blk * BJ, (blk + 1) * BJ)
            xj = [jd_ref[BI * c:BI * c + BI, sl] for c in range(3)]
            nj = [jd_ref[BI * (3 + c):BI * (4 + c), sl] for c in range(3)]
            dx = [xj[c] - xi[c] for c in range(3)]
            ndot = ni[0] * nj[0] + ni[1] * nj[1] + ni[2] * nj[2]
            sq = dx[0] * dx[0] + dx[1] * dx[1] + dx[2] * dx[2]
            t = 2.0 - ndot
            ws.append(jnp.exp(-(sq * t * t)))
            x_loc = [frame[k][0] * dx[0] + frame[k][1] * dx[1]
                     + frame[k][2] * dx[2] for k in range(3)]
            xss.append(jnp.concatenate(
                [x.astype(jnp.bfloat16) for x in x_loc] + [ones_b], axis=0))
        cs = [jnp.maximum(
            jnp.dot(w1k, xs, preferred_element_type=jnp.float32), 0.0)
            for xs in xss]
        hss = [jnp.maximum(
            jnp.dot(w2k, jnp.concatenate(
                [c.astype(jnp.bfloat16), ones_b], axis=0),
                preferred_element_type=jnp.float32), 0.0)
            for c in cs]
        accs = [jnp.zeros((BI, BJ), jnp.float32) for _ in range(8)]
        for blk in range(nchunk):
            sl = slice(blk * BJ, (blk + 1) * BJ)
            w = ws[blk]
            hs = hss[blk]
            for h in range(8):
                hh = hs[BI * h:BI * h + BI, :]
                accs[h] = accs[h] + (w * hh) * jd_ref[BI * (6 + h):BI * (7 + h), sl]
        blk_out = jnp.concatenate(
            [jnp.sum(a, axis=1, keepdims=True) for a in accs], axis=1)
        pair_ref[pl.ds((pid - 1) * BI, BI), :] = blk_out

    @pl.when(pid == NBLK + 1)
    def _epilogue():
        pT = lax.dot_general(eye8_ref[...], pair_ref[...], _CT1,
                             preferred_element_type=jnp.float32)
        f = jnp.dot(w_out1_ref[...], pT,
                    preferred_element_type=jnp.float32) + b_out1_ref[...]
        f = _lrelu(f)
        f = jnp.dot(w_out2_ref[...], f,
                    preferred_element_type=jnp.float32) + b_out2_ref[...]
        f = _lrelu(f)
        f = _group_norm_rows(f, g_out_ref[...], be_out_ref[...], groups=4)
        out_ref[...] = lax.dot_general(
            f, eye16_ref[...], (((0,), (0,)), ((), ())),
            preferred_element_type=jnp.float32)


def kernel(points, nuv, features, W_in1, b_in1, W_in2, b_in2, g_in, be_in,
           Wc1, bc1, Wc2, bc2, W_out1, b_out1, W_out2, b_out2, g_out, be_out):
    feat = features[0]
    pts = points[0]
    nuv9 = nuv[0].reshape(N, 9)

    selx = jnp.eye(3, dtype=jnp.float32) * SCALE
    seln = jnp.eye(3, 9, dtype=jnp.float32)
    eye8 = jnp.eye(8, dtype=jnp.float32)
    eye16 = jnp.eye(16, dtype=jnp.float32)
    eyeb = jnp.eye(BI, dtype=jnp.float32)
    w1k = jnp.kron(jnp.concatenate([Wc1, bc1[:, None]], axis=1),
                   eyeb).astype(jnp.bfloat16)
    w2k = jnp.kron(jnp.concatenate([Wc2, bc2[:, None]], axis=1),
                   eyeb).astype(jnp.bfloat16)

    full = lambda shape: pl.BlockSpec(shape, lambda i: tuple(0 for _ in shape))
    blocked = lambda w: pl.BlockSpec(
        (BI, w), lambda i: (jnp.clip(i - 1, 0, NBLK - 1), 0))
    smem = pl.BlockSpec(memory_space=pltpu.SMEM)

    out = pl.pallas_call(
        _fused_body,
        grid=(NBLK + 2,),
        in_specs=[
            full((N, 16)),
            full((N, 3)),
            full((N, 9)),
            blocked(3),
            blocked(9),
            full((3, 3)), full((3, 9)), full((8, 8)), full((16, 16)),
            full((8, 16)), full((8, 1)), full((8, 8)), full((8, 1)),
            full((8, 1)), full((8, 1)),
            full((8 * BI, 4 * BI)), full((8 * BI, 9 * BI)),
            full((16, 8)), full((16, 1)), full((16, 16)), full((16, 1)),
            full((16, 1)), full((16, 1)),
        ],
        out_specs=pl.BlockSpec((N, 16), lambda i: (0, 0)),
        out_shape=jax.ShapeDtypeStruct((N, 16), jnp.float32),
        scratch_shapes=[
            pltpu.VMEM((14 * BI, N), jnp.float32),
            pltpu.VMEM((N, 8), jnp.float32),
        ],
        compiler_params=pltpu.CompilerParams(
            dimension_semantics=("arbitrary",)),
    )(feat, pts, nuv9, pts, nuv9,
      selx, seln, eye8, eye16,
      W_in1, b_in1.reshape(-1, 1), W_in2, b_in2.reshape(-1, 1),
      g_in.reshape(-1, 1), be_in.reshape(-1, 1),
      w1k, w2k,
      W_out1, b_out1.reshape(-1, 1), W_out2, b_out2.reshape(-1, 1),
      g_out.reshape(-1, 1), be_out.reshape(-1, 1))

    return out[None]


# speedup vs baseline: 1.1297x; 1.1297x over previous
import math

import jax
import jax.numpy as jnp
from jax import lax
from jax.experimental import pallas as pl
from jax.experimental.pallas import tpu as pltpu

N = 2048
BI = 16
BJ = 256
NBLK = N // BI
SCALE = 1.0 / math.sqrt(2.0)

_CT1 = (((1,), (1,)), ((), ()))


def _lrelu(x):
    return jnp.where(x >= 0, x, 0.2 * x)


def _group_norm_rows(x, gamma, beta, groups, eps=1e-5):
    c = x.shape[0]
    per = c // groups
    outs = []
    for g in range(groups):
        sub = x[g * per:(g + 1) * per, :]
        m = jnp.mean(sub)
        v = jnp.mean((sub - m) ** 2)
        outs.append((sub - m) / jnp.sqrt(v + eps))
    y = jnp.concatenate(outs, axis=0)
    return y * gamma + beta


def _fused_body(feat_ref, pts_full_ref, nuv_full_ref, pts_blk_ref, nuv_blk_ref,
                selx_ref, seln_ref, eye8_ref, eye16_ref,
                w_in1_ref, b_in1_ref, w_in2_ref, b_in2_ref, g_in_ref, be_in_ref,
                w1k_ref, w2k_ref,
                w_out1_ref, b_out1_ref, w_out2_ref, b_out2_ref,
                g_out_ref, be_out_ref,
                out_ref, jd_ref, pair_ref):
    pid = pl.program_id(0)

    @pl.when(pid == 0)
    def _prologue():
        f = lax.dot_general(w_in1_ref[...], feat_ref[...], _CT1,
                            preferred_element_type=jnp.float32) + b_in1_ref[...]
        f = _lrelu(f)
        f = jnp.dot(w_in2_ref[...], f,
                    preferred_element_type=jnp.float32) + b_in2_ref[...]
        f = _lrelu(f)
        f = _group_norm_rows(f, g_in_ref[...], be_in_ref[...], groups=4)
        xt = lax.dot_general(selx_ref[...], pts_full_ref[...], _CT1,
                             preferred_element_type=jnp.float32)
        nt = lax.dot_general(seln_ref[...], nuv_full_ref[...], _CT1,
                             preferred_element_type=jnp.float32)
        rows = ([xt[c:c + 1, :] for c in range(3)]
                + [nt[c:c + 1, :] for c in range(3)]
                + [f[h:h + 1, :] for h in range(8)])
        jd_ref[...] = jnp.concatenate(
            [jnp.broadcast_to(r, (BI, N)) for r in rows], axis=0)

    @pl.when((pid >= 1) & (pid <= NBLK))
    def _pair():
        xi = [pts_blk_ref[:, c:c + 1] * SCALE for c in range(3)]
        frame = [[nuv_blk_ref[:, 3 * k + c:3 * k + c + 1] for c in range(3)]
                 for k in range(3)]
        ni = frame[0]
        w1k = w1k_ref[...]
        w2k = w2k_ref[...]
        ones_b = jnp.ones((BI, BJ), jnp.bfloat16)
        nchunk = N // BJ
        ws, xss = [], []
        for blk in range(nchunk):
            sl = slice(blk * BJ, (blk + 1) * BJ)
            xj = [jd_ref[BI * c:BI * c + BI, sl] for c in range(3)]
            nj = [jd_ref[BI * (3 + c):BI * (4 + c), sl] for c in range(3)]
            dx = [xj[c] - xi[c] for c in range(3)]
            ndot = ni[0] * nj[0] + ni[1] * nj[1] + ni[2] * nj[2]
            sq = dx[0] * dx[0] + dx[1] * dx[1] + dx[2] * dx[2]
            t = 2.0 - ndot
            ws.append(jnp.exp(-(sq * t * t)))
            x_loc = [frame[k][0] * dx[0] + frame[k][1] * dx[1]
                     + frame[k][2] * dx[2] for k in range(3)]
            xss.append(jnp.concatenate(
                [x.astype(jnp.bfloat16) for x in x_loc] + [ones_b], axis=0))
        cs = [jnp.maximum(
            jnp.dot(w1k, xs, preferred_element_type=jnp.float32), 0.0)
            for xs in xss]
        hss = [jnp.maximum(
            jnp.dot(w2k, jnp.concatenate(
                [c.astype(jnp.bfloat16), ones_b], axis=0),
                preferred_element_type=jnp.float32), 0.0)
            for c in cs]
        accs = [jnp.zeros((BI, BJ), jnp.float32) for _ in range(8)]
        for blk in range(nchunk):
            sl = slice(blk * BJ, (blk + 1) * BJ)
            w = ws[blk]
            hs = hss[blk]
            for h in range(8):
                hh = hs[BI * h:BI * h + BI, :]
                accs[h] = accs[h] + (w * hh) * jd_ref[BI * (6 + h):BI * (7 + h), sl]
        blk_out = jnp.concatenate(
            [jnp.sum(a, axis=1, keepdims=True) for a in accs], axis=1)
        pair_ref[pl.ds((pid - 1) * BI, BI), :] = blk_out

    @pl.when(pid == NBLK + 1)
    def _epilogue():
        pT = lax.dot_general(eye8_ref[...], pair_ref[...], _CT1,
                             preferred_element_type=jnp.float32)
        f = jnp.dot(w_out1_ref[...], pT,
                    preferred_element_type=jnp.float32) + b_out1_ref[...]
        f = _lrelu(f)
        f = jnp.dot(w_out2_ref[...], f,
                    preferred_element_type=jnp.float32) + b_out2_ref[...]
        f = _lrelu(f)
        f = _group_norm_rows(f, g_out_ref[...], be_out_ref[...], groups=4)
        out_ref[...] = lax.dot_general(
            f, eye16_ref[...], (((0,), (0,)), ((), ())),
            preferred_element_type=jnp.float32)


def kernel(points, nuv, features, W_in1, b_in1, W_in2, b_in2, g_in, be_in,
           Wc1, bc1, Wc2, bc2, W_out1, b_out1, W_out2, b_out2, g_out, be_out):
    feat = features[0]
    pts = points[0]
    nuv9 = nuv[0].reshape(N, 9)

    selx = jnp.eye(3, dtype=jnp.float32) * SCALE
    seln = jnp.eye(3, 9, dtype=jnp.float32)
    eye8 = jnp.eye(8, dtype=jnp.float32)
    eye16 = jnp.eye(16, dtype=jnp.float32)
    eyeb = jnp.eye(BI, dtype=jnp.float32)
    w1k = jnp.kron(jnp.concatenate([Wc1, bc1[:, None]], axis=1),
                   eyeb).astype(jnp.bfloat16)
    w2k = jnp.kron(jnp.concatenate([Wc2, bc2[:, None]], axis=1),
                   eyeb).astype(jnp.bfloat16)

    full = lambda shape: pl.BlockSpec(shape, lambda i: tuple(0 for _ in shape))
    blocked = lambda w: pl.BlockSpec(
        (BI, w), lambda i: (jnp.clip(i - 1, 0, NBLK - 1), 0))
    smem = pl.BlockSpec(memory_space=pltpu.SMEM)

    out = pl.pallas_call(
        _fused_body,
        grid=(NBLK + 2,),
        in_specs=[
            full((N, 16)),
            full((N, 3)),
            full((N, 9)),
            blocked(3),
            blocked(9),
            full((3, 3)), full((3, 9)), full((8, 8)), full((16, 16)),
            full((8, 16)), full((8, 1)), full((8, 8)), full((8, 1)),
            full((8, 1)), full((8, 1)),
            full((8 * BI, 4 * BI)), full((8 * BI, 9 * BI)),
            full((16, 8)), full((16, 1)), full((16, 16)), full((16, 1)),
            full((16, 1)), full((16, 1)),
        ],
        out_specs=pl.BlockSpec((N, 16), lambda i: (0, 0)),
        out_shape=jax.ShapeDtypeStruct((N, 16), jnp.float32),
        scratch_shapes=[
            pltpu.VMEM((14 * BI, N), jnp.float32),
            pltpu.VMEM((N, 8), jnp.float32),
        ],
        compiler_params=pltpu.CompilerParams(
            dimension_semantics=("arbitrary",)),
    )(feat, pts, nuv9, pts, nuv9,
      selx, seln, eye8, eye16,
      W_in1, b_in1.reshape(-1, 1), W_in2, b_in2.reshape(-1, 1),
      g_in.reshape(-1, 1), be_in.reshape(-1, 1),
      w1k, w2k,
      W_out1, b_out1.reshape(-1, 1), W_out2, b_out2.reshape(-1, 1),
      g_out.reshape(-1, 1), be_out.reshape(-1, 1))

    return out[None]
